# baseline (speedup 1.0000x reference)
import functools

import jax
import jax.numpy as jnp
from jax import lax
from jax.experimental import pallas as pl
from jax.experimental.pallas import tpu as pltpu
from jax.experimental.pallas import tpu_sc as plsc

_D = 64


@functools.cache
def _build(B: int):
    info = plsc.get_sparse_core_info()
    nw = info.num_cores * info.num_subcores
    b_per_w = B // nw
    CH = 128
    n_ch = b_per_w // CH
    assert b_per_w % CH == 0 and B % nw == 0
    mesh = plsc.VectorSubcoreMesh(core_axis_name="c", subcore_axis_name="s")

    @functools.partial(
        pl.kernel,
        mesh=mesh,
        out_type=jax.ShapeDtypeStruct((B, _D), jnp.float32),
        scratch_types=[
            pltpu.VMEM((CH,), jnp.int32),
            pltpu.VMEM((CH, _D), jnp.float32),
            pltpu.SemaphoreType.DMA,
        ],
        compiler_params=pltpu.CompilerParams(use_tc_tiling_on_sc=False),
    )
    def k(table_hbm, ids_hbm, out_hbm, idx_v, rows_v, sem):
        wid = lax.axis_index("s") * info.num_cores + lax.axis_index("c")
        base0 = wid * b_per_w

        def body(g, carry):
            base = base0 + g * CH
            pltpu.sync_copy(ids_hbm.at[pl.ds(base, CH)], idx_v)
            pltpu.async_copy(table_hbm.at[idx_v], rows_v, sem).wait()
            pltpu.sync_copy(rows_v, out_hbm.at[pl.ds(base, CH)])
            return carry

        lax.fori_loop(0, n_ch, body, 0)

    return k


def kernel(tone_ids, embed_weight):
    b, s = tone_ids.shape
    ids = tone_ids.reshape(-1).astype(jnp.int32)
    out = _build(b * s)(embed_weight, ids)
    return out.reshape(b, s, _D)


# speedup vs baseline: 1.0024x; 1.0024x over previous
import functools

import jax
import jax.numpy as jnp
from jax import lax
from jax.experimental import pallas as pl
from jax.experimental.pallas import tpu as pltpu
from jax.experimental.pallas import tpu_sc as plsc

_D = 64
_CH = 128
_GPB = 4
_BUF = _CH * _GPB


@functools.cache
def _build(B: int):
    info = plsc.get_sparse_core_info()
    nw = info.num_cores * info.num_subcores
    b_per_w = B // nw
    n_ch = b_per_w // _CH
    n_it = b_per_w // (2 * _BUF)
    assert B % nw == 0 and b_per_w % (2 * _BUF) == 0
    mesh = plsc.VectorSubcoreMesh(core_axis_name="c", subcore_axis_name="s")

    @functools.partial(
        pl.kernel,
        mesh=mesh,
        out_type=jax.ShapeDtypeStruct((B, _D), jnp.float32),
        scratch_types=[
            pltpu.VMEM((n_ch, _CH), jnp.int32),
            pltpu.VMEM((2, _BUF, _D), jnp.float32),
            pltpu.SemaphoreType.DMA,
            pltpu.SemaphoreType.DMA,
            pltpu.SemaphoreType.DMA,
            pltpu.SemaphoreType.DMA,
        ],
        compiler_params=pltpu.CompilerParams(use_tc_tiling_on_sc=False),
    )
    def k(table_hbm, ids_hbm, out_hbm, ids_v, rows_v, g0, g1, o0, o1):
        gsem = (g0, g1)
        osem = (o0, o1)
        wid = lax.axis_index("s") * info.num_cores + lax.axis_index("c")
        base0 = wid * b_per_w
        pltpu.sync_copy(ids_hbm.at[pl.ds(wid * n_ch, n_ch)], ids_v)

        def body(g, carry):
            for b in range(2):
                c = 2 * g + b
                out_slice = out_hbm.at[pl.ds(base0 + c * _BUF, _BUF)]

                @pl.when(g > 0)
                def _wait_prev():
                    pltpu.make_async_copy(rows_v.at[b], out_slice,
                                          osem[b]).wait()

                for j in range(_GPB):
                    pltpu.make_async_copy(
                        table_hbm.at[ids_v.at[c * _GPB + j]],
                        rows_v.at[b, pl.ds(j * _CH, _CH)], gsem[b]).start()
                for j in range(_GPB):
                    pltpu.make_async_copy(
                        table_hbm.at[ids_v.at[c * _GPB + j]],
                        rows_v.at[b, pl.ds(j * _CH, _CH)], gsem[b]).wait()
                pltpu.make_async_copy(rows_v.at[b], out_slice,
                                      osem[b]).start()
            return carry

        lax.fori_loop(0, n_it, body, 0)
        for b in range(2):
            c = 2 * (n_it - 1) + b
            out_slice = out_hbm.at[pl.ds(base0 + c * _BUF, _BUF)]
            pltpu.make_async_copy(rows_v.at[b], out_slice, osem[b]).wait()

    return k


def kernel(tone_ids, embed_weight):
    b, s = tone_ids.shape
    n = b * s
    ids = tone_ids.reshape(n // _CH, _CH).astype(jnp.int32)
    out = _build(n)(embed_weight, ids)
    return out.reshape(b, s, _D)


# speedup vs baseline: 2.5558x; 2.5498x over previous
import functools

import jax
import jax.numpy as jnp
from jax import lax
from jax.experimental import pallas as pl
from jax.experimental.pallas import tpu as pltpu
from jax.experimental.pallas import tpu_sc as plsc

_D = 64
_V = 6
_BUF = 512
_GRP = 16


@functools.cache
def _build(B: int):
    info = plsc.get_sparse_core_info()
    nw = info.num_cores * info.num_subcores
    b_per_w = B // nw
    n_it = b_per_w // (2 * _BUF)
    n_grp = _BUF // _GRP
    assert B % nw == 0 and b_per_w % (2 * _BUF) == 0
    mesh = plsc.VectorSubcoreMesh(core_axis_name="c", subcore_axis_name="s")

    @functools.partial(
        pl.kernel,
        mesh=mesh,
        out_type=jax.ShapeDtypeStruct((B * _D,), jnp.float32),
        scratch_types=[
            pltpu.VMEM((_V * _D,), jnp.float32),
            pltpu.VMEM((b_per_w,), jnp.int32),
            pltpu.VMEM((2, _BUF * _D), jnp.float32),
            pltpu.SemaphoreType.DMA,
            pltpu.SemaphoreType.DMA,
        ],
        compiler_params=pltpu.CompilerParams(use_tc_tiling_on_sc=False,
                                             needs_layout_passes=False),
    )
    def k(tflat_hbm, ids_hbm, out_hbm, tflat_v, ids_v, rows_v, o0, o1):
        osem = (o0, o1)
        wid = lax.axis_index("s") * info.num_cores + lax.axis_index("c")
        base0 = wid * b_per_w
        pltpu.sync_copy(tflat_hbm, tflat_v)
        pltpu.sync_copy(ids_hbm.at[pl.ds(base0, b_per_w)], ids_v)
        iota = lax.iota(jnp.int32, _GRP)
        iota64 = iota * _D

        def body(g, carry):
            for b in range(2):
                c = 2 * g + b
                out_slice = out_hbm.at[pl.ds((base0 + c * _BUF) * _D,
                                             _BUF * _D)]

                @pl.when(g > 0)
                def _wait_prev():
                    pltpu.make_async_copy(rows_v.at[b], out_slice,
                                          osem[b]).wait()

                def grp(i, carry2):
                    v_ids = ids_v[pl.ds(c * _BUF + i * _GRP, _GRP)]
                    v_g0 = v_ids * _D
                    v_s0 = iota64 + i * (_GRP * _D)
                    for d in range(_D):
                        vals = plsc.load_gather(tflat_v, [v_g0 + d])
                        plsc.store_scatter(rows_v.at[b], [v_s0 + d], vals)
                    return carry2

                lax.fori_loop(0, n_grp, grp, 0)
                pltpu.make_async_copy(rows_v.at[b], out_slice,
                                      osem[b]).start()
            return carry

        lax.fori_loop(0, n_it, body, 0)
        for b in range(2):
            c = 2 * (n_it - 1) + b
            out_slice = out_hbm.at[pl.ds((base0 + c * _BUF) * _D, _BUF * _D)]
            pltpu.make_async_copy(rows_v.at[b], out_slice, osem[b]).wait()

    return k


def kernel(tone_ids, embed_weight):
    b, s = tone_ids.shape
    n = b * s
    ids = tone_ids.reshape(-1).astype(jnp.int32)
    out = _build(n)(embed_weight.reshape(-1), ids)
    return out.reshape(b, s, _D)


# speedup vs baseline: 10.6396x; 4.1628x over previous
import functools

import jax
import jax.numpy as jnp
from jax import lax
from jax.experimental import pallas as pl
from jax.experimental.pallas import tpu as pltpu
from jax.experimental.pallas import tpu_sc as plsc

_D = 64
_V = 6
_BUF = 320
_GRP = 16


@functools.cache
def _build(B: int):
    info = plsc.get_sparse_core_info()
    nw = info.num_cores * info.num_subcores
    b_per_w = B // nw
    n_it = b_per_w // (2 * _BUF)
    n_grp = _BUF // _GRP
    assert B % nw == 0 and b_per_w % (2 * _BUF) == 0
    mesh = plsc.VectorSubcoreMesh(core_axis_name="c", subcore_axis_name="s")

    @functools.partial(
        pl.kernel,
        mesh=mesh,
        out_type=jax.ShapeDtypeStruct((B, _D), jnp.float32),
        scratch_types=[
            pltpu.VMEM((_V * _D,), jnp.float32),
            pltpu.VMEM((b_per_w,), jnp.int32),
            pltpu.VMEM((2, _BUF, _D), jnp.float32),
            pltpu.SemaphoreType.DMA,
            pltpu.SemaphoreType.DMA,
        ],
        compiler_params=pltpu.CompilerParams(needs_layout_passes=False),
    )
    def k(tflat_hbm, ids_hbm, out_hbm, tflat_v, ids_v, rows_v, o0, o1):
        osem = (o0, o1)
        wid = lax.axis_index("s") * info.num_cores + lax.axis_index("c")
        base0 = wid * b_per_w
        pltpu.sync_copy(tflat_hbm, tflat_v)
        pltpu.sync_copy(ids_hbm.at[pl.ds(base0, b_per_w)], ids_v)
        iota = lax.iota(jnp.int32, _GRP)

        def body(g, carry):
            for b in range(2):
                c = 2 * g + b
                out_slice = out_hbm.at[pl.ds(base0 + c * _BUF, _BUF)]

                @pl.when(g > 0)
                def _wait_prev():
                    pltpu.make_async_copy(rows_v.at[b], out_slice,
                                          osem[b]).wait()

                def grp(i, carry2):
                    v_ids = ids_v[pl.ds(c * _BUF + i * _GRP, _GRP)]
                    v_off = v_ids * _D
                    for r in range(_GRP):
                        bc = lax.gather(
                            v_off,
                            jnp.full((_GRP, 1), r, jnp.int32),
                            lax.GatherDimensionNumbers(
                                offset_dims=(), collapsed_slice_dims=(0,),
                                start_index_map=(0,)),
                            (1,),
                            mode=lax.GatherScatterMode.PROMISE_IN_BOUNDS)
                        row = i * _GRP + r
                        for j in range(_D // _GRP):
                            vals = plsc.load_gather(
                                tflat_v, [bc + (iota + j * _GRP)])
                            rows_v[b, row, pl.ds(j * _GRP, _GRP)] = vals
                    return carry2

                lax.fori_loop(0, n_grp, grp, 0)
                pltpu.make_async_copy(rows_v.at[b], out_slice,
                                      osem[b]).start()
            return carry

        lax.fori_loop(0, n_it, body, 0)
        for b in range(2):
            c = 2 * (n_it - 1) + b
            out_slice = out_hbm.at[pl.ds(base0 + c * _BUF, _BUF)]
            pltpu.make_async_copy(rows_v.at[b], out_slice, osem[b]).wait()

    return k


def kernel(tone_ids, embed_weight):
    b, s = tone_ids.shape
    n = b * s
    ids = tone_ids.reshape(-1).astype(jnp.int32)
    out = _build(n)(embed_weight.reshape(-1), ids)
    return out.reshape(b, s, _D)
